# baseline (speedup 1.0000x reference)
import functools

import jax
import jax.numpy as jnp
from jax import lax
from jax.experimental import pallas as pl
from jax.experimental.pallas import tpu as pltpu
from jax.experimental.pallas import tpu_sc as plsc

B = 16384
NF = 32
NV = 1000000
NC = 2
NS = 16
NW = NC * NS
BPW = B // NW
L = 16
CHUNK = 128
NCH = BPW // CHUNK
KSTRIDE = NV // 8


def _make_sc_call():
    mesh = plsc.VectorSubcoreMesh(core_axis_name="c", subcore_axis_name="s")

    @functools.partial(
        pl.kernel,
        mesh=mesh,
        compiler_params=pltpu.CompilerParams(
            needs_layout_passes=False, use_tc_tiling_on_sc=False),
        out_type=jax.ShapeDtypeStruct((B,), jnp.float32),
        scratch_types=[
            pltpu.VMEM((NCH, CHUNK), jnp.int32),
            pltpu.VMEM((NCH, CHUNK), jnp.int32),
            pltpu.VMEM((NF, CHUNK), jnp.int32),
            pltpu.VMEM((NF, CHUNK), jnp.int32),
            pltpu.VMEM((NF, CHUNK, 8), jnp.float32),
            pltpu.VMEM((NF, CHUNK, 8), jnp.float32),
            pltpu.VMEM((BPW,), jnp.float32),
            pltpu.SemaphoreType.DMA,
            pltpu.SemaphoreType.DMA,
        ],
    )
    def sc_kernel(users_hbm, items_hbm, ut8_hbm, it8_hbm, out_hbm,
                  uidx_v, iidx_v, urow_v, irow_v, usl_v, isl_v, out_v,
                  sem_u, sem_i):
        wid = lax.axis_index("s") * NC + lax.axis_index("c")
        base = wid * BPW

        pltpu.sync_copy(users_hbm.at[wid], uidx_v)
        pltpu.sync_copy(items_hbm.at[wid], iidx_v)

        lane = lax.iota(jnp.int32, L)

        def do_chunk(c, carry):
            def rows_for(i, carry2):
                uvec = uidx_v[c, pl.ds(i * L, L)] >> 3
                ivec = iidx_v[c, pl.ds(i * L, L)] >> 3
                for k in range(NF):
                    urow_v[k, pl.ds(i * L, L)] = uvec + (k * KSTRIDE)
                    irow_v[k, pl.ds(i * L, L)] = ivec + (k * KSTRIDE)
                return carry2

            lax.fori_loop(0, CHUNK // L, rows_for, 0)

            copies = []
            for k in range(NF):
                copies.append(pltpu.async_copy(
                    ut8_hbm.at[urow_v.at[k]], usl_v.at[k], sem_u))
                copies.append(pltpu.async_copy(
                    it8_hbm.at[irow_v.at[k]], isl_v.at[k], sem_i))
            for cp in copies:
                cp.wait()

            def dot16(i, carry2):
                uoff = uidx_v[c, pl.ds(i * L, L)] & 7
                ioff = iidx_v[c, pl.ds(i * L, L)] & 7
                pvec = lane + i * L
                acc = jnp.zeros((L,), jnp.float32)
                for k in range(NF):
                    kvec = jnp.full((L,), k, jnp.int32)
                    u = plsc.load_gather(usl_v, [kvec, pvec, uoff])
                    v = plsc.load_gather(isl_v, [kvec, pvec, ioff])
                    acc = acc + u * v
                out_v[pl.ds(c * CHUNK + i * L, L)] = acc
                return carry2

            lax.fori_loop(0, CHUNK // L, dot16, 0)
            return carry

        lax.fori_loop(0, NCH, do_chunk, 0)

        pltpu.sync_copy(out_v, out_hbm.at[pl.ds(base, BPW)])

    return sc_kernel


_sc_call = _make_sc_call()


@jax.jit
def kernel(data, user_factors, item_factors):
    data = data.astype(jnp.int32)
    users = data[:, 0].reshape(NW, NCH, CHUNK)
    items = data[:, 1].reshape(NW, NCH, CHUNK)
    ut8 = user_factors.T.reshape(NF * KSTRIDE, 8)
    it8 = item_factors.T.reshape(NF * KSTRIDE, 8)
    return _sc_call(users, items, ut8, it8)


# speedup vs baseline: 1.0448x; 1.0448x over previous
import functools

import jax
import jax.numpy as jnp
from jax import lax
from jax.experimental import pallas as pl
from jax.experimental.pallas import tpu as pltpu
from jax.experimental.pallas import tpu_sc as plsc

B = 16384
NF = 32
NV = 1000000
NC = 2
NS = 16
NW = NC * NS
BPW = B // NW
L = 16
CHUNK = 128
NCH = BPW // CHUNK


def _make_sc_call():
    mesh = plsc.VectorSubcoreMesh(core_axis_name="c", subcore_axis_name="s")

    @functools.partial(
        pl.kernel,
        mesh=mesh,
        compiler_params=pltpu.CompilerParams(
            needs_layout_passes=False, use_tc_tiling_on_sc=False),
        out_type=jax.ShapeDtypeStruct((B,), jnp.float32),
        scratch_types=[
            pltpu.VMEM((NCH, CHUNK), jnp.int32),
            pltpu.VMEM((NCH, CHUNK), jnp.int32),
            pltpu.VMEM((CHUNK,), jnp.int32),
            pltpu.VMEM((CHUNK,), jnp.int32),
            pltpu.VMEM((NF, CHUNK, 8), jnp.float32),
            pltpu.VMEM((NF, CHUNK, 8), jnp.float32),
            pltpu.VMEM((BPW,), jnp.float32),
            pltpu.SemaphoreType.DMA,
            pltpu.SemaphoreType.DMA,
        ],
    )
    def sc_kernel(users_hbm, items_hbm, ut3_hbm, it3_hbm, out_hbm,
                  uidx_v, iidx_v, urow_v, irow_v, usl_v, isl_v, out_v,
                  sem_u, sem_i):
        wid = lax.axis_index("s") * NC + lax.axis_index("c")
        base = wid * BPW

        pltpu.sync_copy(users_hbm.at[wid], uidx_v)
        pltpu.sync_copy(items_hbm.at[wid], iidx_v)

        lane = lax.iota(jnp.int32, L)

        def do_chunk(c, carry):
            def rows_for(i, carry2):
                urow_v[pl.ds(i * L, L)] = uidx_v[c, pl.ds(i * L, L)] >> 3
                irow_v[pl.ds(i * L, L)] = iidx_v[c, pl.ds(i * L, L)] >> 3
                return carry2

            lax.fori_loop(0, CHUNK // L, rows_for, 0)

            copies = []
            for k in range(NF):
                copies.append(pltpu.async_copy(
                    ut3_hbm.at[k].at[urow_v], usl_v.at[k], sem_u))
                copies.append(pltpu.async_copy(
                    it3_hbm.at[k].at[irow_v], isl_v.at[k], sem_i))
            for cp in copies:
                cp.wait()

            def dot16(i, carry2):
                uoff = uidx_v[c, pl.ds(i * L, L)] & 7
                ioff = iidx_v[c, pl.ds(i * L, L)] & 7
                pvec = lane + i * L
                acc = jnp.zeros((L,), jnp.float32)
                for k in range(NF):
                    kvec = jnp.full((L,), k, jnp.int32)
                    u = plsc.load_gather(usl_v, [kvec, pvec, uoff])
                    v = plsc.load_gather(isl_v, [kvec, pvec, ioff])
                    acc = acc + u * v
                out_v[pl.ds(c * CHUNK + i * L, L)] = acc
                return carry2

            lax.fori_loop(0, CHUNK // L, dot16, 0)
            return carry

        lax.fori_loop(0, NCH, do_chunk, 0)

        pltpu.sync_copy(out_v, out_hbm.at[pl.ds(base, BPW)])

    return sc_kernel


_sc_call = _make_sc_call()


@jax.jit
def kernel(data, user_factors, item_factors):
    data = data.astype(jnp.int32)
    users = data[:, 0].reshape(NW, NCH, CHUNK)
    items = data[:, 1].reshape(NW, NCH, CHUNK)
    ut3 = user_factors.T.reshape(NF, NV // 8, 8)
    it3 = item_factors.T.reshape(NF, NV // 8, 8)
    return _sc_call(users, items, ut3, it3)
